# baseline (speedup 1.0000x reference)
import functools

import jax
import jax.numpy as jnp
from jax import lax
from jax.experimental import pallas as pl
from jax.experimental.pallas import tpu as pltpu
from jax.experimental.pallas import tpu_sc as plsc

B, S, D = 4, 8192, 1024
NC, NS, L = 2, 16, 16
NW = NC * NS
ROWS_PER_W = S // NW
BLK = 32
NBLK = ROWS_PER_W // BLK
BLK_ELEMS = BLK * D


def _sc_body(x_hbm, t_hbm, o_hbm, x_v, t_v):
    wid = lax.axis_index("s") * NC + lax.axis_index("c")
    base_row = wid * ROWS_PER_W
    for blk in range(NBLK):
        t_off = (base_row + blk * BLK) * D
        pltpu.sync_copy(t_hbm.at[pl.ds(t_off, BLK_ELEMS)], t_v)
        for b in range(B):
            x_off = (b * S + base_row + blk * BLK) * D
            pltpu.sync_copy(x_hbm.at[pl.ds(x_off, BLK_ELEMS)], x_v)

            def body(i, _):
                sl = pl.ds(i * (8 * L), 8 * L)
                for j in range(8):
                    sj = pl.ds(i * (8 * L) + j * L, L)
                    x_v[sj] = x_v[sj] + t_v[sj]
                return 0

            lax.fori_loop(0, BLK_ELEMS // (8 * L), body, 0)
            pltpu.sync_copy(x_v, o_hbm.at[pl.ds(x_off, BLK_ELEMS)])


def kernel(inputs, pos_table):
    mesh = plsc.VectorSubcoreMesh(core_axis_name="c", subcore_axis_name="s")
    sc_add = functools.partial(
        pl.kernel,
        mesh=mesh,
        out_type=jax.ShapeDtypeStruct((B * S * D,), jnp.float32),
        scratch_types=[
            pltpu.VMEM((BLK_ELEMS,), jnp.float32),
            pltpu.VMEM((BLK_ELEMS,), jnp.float32),
        ],
    )(_sc_body)
    out = sc_add(inputs.reshape(B * S * D), pos_table.reshape(S * D))
    return out.reshape(B, S, D)
